# baseline (speedup 1.0000x reference)
import functools

import jax
import jax.numpy as jnp
from jax import lax
from jax.experimental import pallas as pl
from jax.experimental.pallas import tpu as pltpu
from jax.experimental.pallas import tpu_sc as plsc

_B = 4
_SEQ = 8192
_L = 16
_NV = _SEQ // _L
_NBC = 256
_NBF = 32


def _make_mask_kernel():
    mesh = plsc.VectorSubcoreMesh(
        core_axis_name="c", subcore_axis_name="s", num_cores=1
    )

    @functools.partial(
        pl.kernel,
        mesh=mesh,
        out_type=jax.ShapeDtypeStruct((_B, _SEQ), jnp.int32),
        scratch_types=[
            pltpu.VMEM((_SEQ,), jnp.int32),
            pltpu.VMEM((_SEQ,), jnp.int32),
            pltpu.VMEM((_NBC,), jnp.int32),
            pltpu.VMEM((_NBF,), jnp.int32),
            pltpu.VMEM((_L,), jnp.int32),
        ],
        compiler_params=pltpu.CompilerParams(needs_layout_passes=False),
    )
    def mask_kernel(orders_hbm, thresh_hbm, out_hbm, row_v, out_v, hist_v, fine_v, th_v):
        s = lax.axis_index("s")
        lane = jnp.arange(_L, dtype=jnp.int32)

        @pl.when(s < _B)
        def _():
            pltpu.sync_copy(orders_hbm.at[s], row_v)
            pltpu.sync_copy(thresh_hbm, th_v)
            t_thresh = jnp.sum(jnp.where(lane == s, th_v[...], 0))

            def zero_hist(i, _):
                hist_v[pl.ds(i * _L, _L)] = jnp.zeros((_L,), jnp.int32)
                return 0

            lax.fori_loop(0, _NBC // _L, zero_hist, 0, unroll=4)

            def coarse_body(i, _):
                x = row_v[pl.ds(i * _L, _L)]
                b = x >> 5
                cnt, last = plsc.scan_count(b)
                plsc.addupdate_scatter(hist_v, [b], cnt, mask=last)
                return 0

            lax.fori_loop(0, _NV, coarse_body, 0, unroll=4)

            def search_body(g, carry):
                found, cstar, pexcl, run = carry
                tot = hist_v[pl.ds(g * _L, _L)]
                incl = plsc.cumsum(tot) + run
                ge = incl >= t_thresh
                ffs = plsc.all_reduce_ffs(ge)
                cnt_ge = plsc.all_reduce_population_count(ge)
                pick = jnp.logical_and(jnp.logical_not(found), cnt_ge > 0)
                excl_here = jnp.sum(jnp.where(lane == ffs, incl - tot, 0))
                cstar = jnp.where(pick, g * _L + ffs, cstar)
                pexcl = jnp.where(pick, excl_here, pexcl)
                found = jnp.logical_or(found, cnt_ge > 0)
                run = run + jnp.sum(tot)
                return found, cstar, pexcl, run

            zero_s = jnp.zeros((_L,), jnp.int32)
            found0 = jnp.zeros((_L,), jnp.bool_)
            _, cstar, pexcl, _ = lax.fori_loop(
                0, _NBC // _L, search_body, (found0, zero_s, zero_s, zero_s)
            )

            fine_v[pl.ds(0, _L)] = jnp.zeros((_L,), jnp.int32)
            fine_v[pl.ds(_L, _L)] = jnp.zeros((_L,), jnp.int32)

            def fine_body(i, _):
                x = row_v[pl.ds(i * _L, _L)]
                m = (x >> 5) == cstar
                f = x & 31
                cnt, last = plsc.scan_count(f, mask=m)
                plsc.addupdate_scatter(
                    fine_v, [f], cnt, mask=jnp.logical_and(last, m)
                )
                return 0

            lax.fori_loop(0, _NV, fine_body, 0, unroll=4)

            def fine_search(g, carry):
                found, vidx, cless, run = carry
                tot = fine_v[pl.ds(g * _L, _L)]
                incl = plsc.cumsum(tot) + run + pexcl
                ge = incl >= t_thresh
                ffs = plsc.all_reduce_ffs(ge)
                cnt_ge = plsc.all_reduce_population_count(ge)
                pick = jnp.logical_and(jnp.logical_not(found), cnt_ge > 0)
                excl_here = jnp.sum(jnp.where(lane == ffs, incl - tot, 0))
                vidx = jnp.where(pick, g * _L + ffs, vidx)
                cless = jnp.where(pick, excl_here, cless)
                found = jnp.logical_or(found, cnt_ge > 0)
                run = run + jnp.sum(tot)
                return found, vidx, cless, run

            _, vidx, cless, _ = lax.fori_loop(
                0, _NBF // _L, fine_search, (found0, zero_s, zero_s, zero_s)
            )
            vstar = cstar * 32 + vidx
            nties = t_thresh - cless

            def mask_body(i, tie_off):
                x = row_v[pl.ds(i * _L, _L)]
                lt = x < vstar
                eq = x == vstar
                eq_i = jnp.where(eq, 1, 0)
                incl = plsc.cumsum(eq_i)
                tie_rank = tie_off + incl - eq_i
                m = jnp.logical_or(lt, jnp.logical_and(eq, tie_rank < nties))
                out_v[pl.ds(i * _L, _L)] = jnp.where(m, 1, 0)
                return tie_off + jnp.sum(eq_i)

            lax.fori_loop(0, _NV, mask_body, jnp.int32(0), unroll=4)
            pltpu.sync_copy(out_v, out_hbm.at[s])

    return mask_kernel


_mask_kernel = _make_mask_kernel()


def kernel(patches, orders):
    batch_size, seq_len, _hidden = patches.shape
    std = 0.25
    mean = 1.0
    a, b = 0.0, 1.0
    rkey = jax.random.key(42)
    lower = (a - mean) / std
    upper = (b - mean) / std
    mask_rates = (
        jax.random.truncated_normal(rkey, lower, upper, (batch_size,), jnp.float32)
        * std
        + mean
    )
    thresholds = jnp.ceil(mask_rates * seq_len).astype(jnp.int32)
    th_padded = jnp.zeros((_L,), jnp.int32).at[:batch_size].set(thresholds)
    out_i32 = _mask_kernel(orders.astype(jnp.int32), th_padded)
    return out_i32.astype(jnp.bool_)


# speedup vs baseline: 1.2220x; 1.2220x over previous
import functools

import jax
import jax.numpy as jnp
from jax import lax
from jax.experimental import pallas as pl
from jax.experimental.pallas import tpu as pltpu
from jax.experimental.pallas import tpu_sc as plsc

_B = 4
_SEQ = 8192
_L = 16
_NV = _SEQ // _L
_NBC = 256
_NBF = 32


def _make_mask_kernel():
    mesh = plsc.VectorSubcoreMesh(
        core_axis_name="c", subcore_axis_name="s", num_cores=1
    )

    @functools.partial(
        pl.kernel,
        mesh=mesh,
        out_type=jax.ShapeDtypeStruct((_B, _SEQ), jnp.int32),
        scratch_types=[
            pltpu.VMEM((_SEQ,), jnp.int32),
            pltpu.VMEM((_SEQ,), jnp.int32),
            pltpu.VMEM((_NBC * _L,), jnp.int32),
            pltpu.VMEM((_NBF * _L,), jnp.int32),
            pltpu.VMEM((_L,), jnp.int32),
        ],
        compiler_params=pltpu.CompilerParams(needs_layout_passes=False),
    )
    def mask_kernel(orders_hbm, thresh_hbm, out_hbm, row_v, out_v, hist_v, fine_v, th_v):
        s = lax.axis_index("s")
        lane = jnp.arange(_L, dtype=jnp.int32)

        @pl.when(s < _B)
        def _():
            pltpu.sync_copy(orders_hbm.at[s], row_v)
            pltpu.sync_copy(thresh_hbm, th_v)
            t_thresh = jnp.sum(jnp.where(lane == s, th_v[...], 0))

            ones = jnp.ones((_L,), jnp.int32)

            def zero_hist(i, _):
                hist_v[pl.ds(i * _L, _L)] = jnp.zeros((_L,), jnp.int32)
                return 0

            lax.fori_loop(0, _NBC, zero_hist, 0, unroll=8)

            def coarse_body(i, _):
                x = row_v[pl.ds(i * _L, _L)]
                idx = ((x >> 5) << 4) | lane
                plsc.addupdate_scatter(hist_v, [idx], ones)
                return 0

            lax.fori_loop(0, _NV, coarse_body, 0, unroll=4)

            def search_body(g, carry):
                found, cstar, pexcl, run = carry
                base = g * _NBC + lane * _L
                tot = jnp.zeros((_L,), jnp.int32)
                for l in range(_L):
                    tot = tot + plsc.load_gather(hist_v, [base + l])
                incl = plsc.cumsum(tot) + run
                ge = incl >= t_thresh
                ffs = plsc.all_reduce_ffs(ge)
                cnt_ge = plsc.all_reduce_population_count(ge)
                pick = jnp.logical_and(jnp.logical_not(found), cnt_ge > 0)
                excl_here = jnp.sum(jnp.where(lane == ffs, incl - tot, 0))
                cstar = jnp.where(pick, g * _L + ffs, cstar)
                pexcl = jnp.where(pick, excl_here, pexcl)
                found = jnp.logical_or(found, cnt_ge > 0)
                run = run + jnp.sum(tot)
                return found, cstar, pexcl, run

            zero_s = jnp.zeros((_L,), jnp.int32)
            found0 = jnp.zeros((_L,), jnp.bool_)
            _, cstar, pexcl, _ = lax.fori_loop(
                0, _NBC // _L, search_body, (found0, zero_s, zero_s, zero_s)
            )

            def zero_fine(i, _):
                fine_v[pl.ds(i * _L, _L)] = jnp.zeros((_L,), jnp.int32)
                return 0

            lax.fori_loop(0, _NBF, zero_fine, 0, unroll=8)

            def fine_body(i, _):
                x = row_v[pl.ds(i * _L, _L)]
                m = (x >> 5) == cstar
                idx = ((x & 31) << 4) | lane
                plsc.addupdate_scatter(fine_v, [idx], ones, mask=m)
                return 0

            lax.fori_loop(0, _NV, fine_body, 0, unroll=4)

            def fine_search(g, carry):
                found, vidx, cless, run = carry
                base = g * _NBC + lane * _L
                tot = jnp.zeros((_L,), jnp.int32)
                for l in range(_L):
                    tot = tot + plsc.load_gather(fine_v, [base + l])
                incl = plsc.cumsum(tot) + run + pexcl
                ge = incl >= t_thresh
                ffs = plsc.all_reduce_ffs(ge)
                cnt_ge = plsc.all_reduce_population_count(ge)
                pick = jnp.logical_and(jnp.logical_not(found), cnt_ge > 0)
                excl_here = jnp.sum(jnp.where(lane == ffs, incl - tot, 0))
                vidx = jnp.where(pick, g * _L + ffs, vidx)
                cless = jnp.where(pick, excl_here, cless)
                found = jnp.logical_or(found, cnt_ge > 0)
                run = run + jnp.sum(tot)
                return found, vidx, cless, run

            _, vidx, cless, _ = lax.fori_loop(
                0, _NBF // _L, fine_search, (found0, zero_s, zero_s, zero_s)
            )
            vstar = cstar * 32 + vidx
            nties = t_thresh - cless

            def mask_body(i, tie_off):
                x = row_v[pl.ds(i * _L, _L)]
                lt = x < vstar
                eq = x == vstar
                eq_i = jnp.where(eq, 1, 0)
                incl = plsc.cumsum(eq_i)
                tie_rank = tie_off + incl - eq_i
                m = jnp.logical_or(lt, jnp.logical_and(eq, tie_rank < nties))
                out_v[pl.ds(i * _L, _L)] = jnp.where(m, 1, 0)
                return tie_off + jnp.sum(eq_i)

            lax.fori_loop(0, _NV, mask_body, jnp.int32(0), unroll=4)
            pltpu.sync_copy(out_v, out_hbm.at[s])

    return mask_kernel


_mask_kernel = _make_mask_kernel()


def kernel(patches, orders):
    batch_size, seq_len, _hidden = patches.shape
    std = 0.25
    mean = 1.0
    a, b = 0.0, 1.0
    rkey = jax.random.key(42)
    lower = (a - mean) / std
    upper = (b - mean) / std
    mask_rates = (
        jax.random.truncated_normal(rkey, lower, upper, (batch_size,), jnp.float32)
        * std
        + mean
    )
    thresholds = jnp.ceil(mask_rates * seq_len).astype(jnp.int32)
    th_padded = jnp.zeros((_L,), jnp.int32).at[:batch_size].set(thresholds)
    out_i32 = _mask_kernel(orders.astype(jnp.int32), th_padded)
    return out_i32.astype(jnp.bool_)


# speedup vs baseline: 1.4133x; 1.1565x over previous
import functools

import jax
import jax.numpy as jnp
from jax import lax
from jax.experimental import pallas as pl
from jax.experimental.pallas import tpu as pltpu
from jax.experimental.pallas import tpu_sc as plsc

_B = 4
_SEQ = 8192
_L = 16
_NV = _SEQ // _L


def _make_mask_kernel():
    nc = 1

    mesh = plsc.VectorSubcoreMesh(
        core_axis_name="c", subcore_axis_name="s", num_cores=nc
    )

    @functools.partial(
        pl.kernel,
        mesh=mesh,
        out_type=jax.ShapeDtypeStruct((_B, _SEQ), jnp.int32),
        scratch_types=[
            pltpu.VMEM((_SEQ,), jnp.int32),
            pltpu.VMEM((_SEQ,), jnp.int32),
            pltpu.VMEM((_L,), jnp.int32),
        ],
        compiler_params=pltpu.CompilerParams(needs_layout_passes=False),
    )
    def mask_kernel(orders_hbm, thresh_hbm, out_hbm, row_v, outrow_v, th_v):
        c = lax.axis_index("c")
        s = lax.axis_index("s")
        wid = s * nc + c

        @pl.when(wid < _B)
        def _():
            pltpu.sync_copy(orders_hbm.at[wid], row_v)
            pltpu.sync_copy(thresh_hbm, th_v)
            lane = jnp.arange(_L, dtype=jnp.int32)
            t_thresh = jnp.sum(jnp.where(lane == wid, th_v[...], 0))

            def count_le(mid):
                def body(i, acc):
                    x = row_v[pl.ds(i * _L, _L)]
                    return acc + jnp.where(x <= mid, 1, 0)

                acc = lax.fori_loop(
                    0, _NV, body, jnp.zeros((_L,), jnp.int32), unroll=8
                )
                return jnp.sum(acc)

            def bs_body(_, carry):
                lo, hi, cless = carry
                mid = (lo + hi) // 2
                cnt = count_le(mid)
                ge = cnt >= t_thresh
                return (
                    jnp.where(ge, lo, mid + 1),
                    jnp.where(ge, mid, hi),
                    jnp.where(ge, cless, cnt),
                )

            vstar, _, count_less = lax.fori_loop(
                0,
                13,
                bs_body,
                (jnp.int32(0), jnp.int32(_SEQ - 1), jnp.int32(0)),
            )
            num_ties = t_thresh - count_less

            def body_mask(i, tie_off):
                x = row_v[pl.ds(i * _L, _L)]
                lt = x < vstar
                eq = x == vstar
                eq_i = jnp.where(eq, 1, 0)
                incl = plsc.cumsum(eq_i)
                tie_rank = tie_off + incl - eq_i
                m = lt | (eq & (tie_rank < num_ties))
                outrow_v[pl.ds(i * _L, _L)] = jnp.where(m, 1, 0)
                return tie_off + jnp.sum(eq_i)

            lax.fori_loop(0, _NV, body_mask, jnp.int32(0), unroll=4)
            pltpu.sync_copy(outrow_v, out_hbm.at[wid])

    return mask_kernel


_mask_kernel = _make_mask_kernel()


def kernel(patches, orders):
    batch_size, seq_len, _hidden = patches.shape
    std = 0.25
    mean = 1.0
    a, b = 0.0, 1.0
    rkey = jax.random.key(42)
    lower = (a - mean) / std
    upper = (b - mean) / std
    mask_rates = (
        jax.random.truncated_normal(rkey, lower, upper, (batch_size,), jnp.float32)
        * std
        + mean
    )
    thresholds = jnp.ceil(mask_rates * seq_len).astype(jnp.int32)
    th_padded = jnp.zeros((_L,), jnp.int32).at[:batch_size].set(thresholds)
    out_i32 = _mask_kernel(orders.astype(jnp.int32), th_padded)
    return out_i32.astype(jnp.bool_)
